# baseline (speedup 1.0000x reference)
import functools

import jax
import jax.numpy as jnp
from jax.experimental import pallas as pl
from jax.experimental.pallas import tpu as pltpu

DQ = 128
DM = 512


def _tc_body(idx_ref, ct_ref, p_ref, sw_ref, sb_ref, w_ref, b_ref,
             o_ref, zt_ref, const_ref, *, h, w, rows):
    i = pl.program_id(0)
    nb = rows * w

    @pl.when(i == 0)
    def _():
        wc = w_ref[0:DQ, :]
        ws = w_ref[2 * DQ:3 * DQ, :]
        zt_ref[...] = jnp.dot(ct_ref[...], wc, preferred_element_type=jnp.float32)
        size_e = (float(h) * sw_ref[0:1, :] + float(w) * sw_ref[1:2, :]
                  + sb_ref[0:1, :])
        const_ref[...] = (jnp.dot(size_e, ws, preferred_element_type=jnp.float32)
                          + b_ref[0:1, :])

    wp = w_ref[DQ:2 * DQ, :]
    lanes = jax.lax.broadcasted_iota(jnp.int32, (1, rows, w, DQ), 3)
    oh = (lanes == idx_ref[...][..., None]).astype(jnp.float32).reshape(nb, DQ)
    acc = jnp.dot(oh, zt_ref[...], preferred_element_type=jnp.float32)
    pos = p_ref[...].reshape(nb, DQ)
    acc = acc + jnp.dot(pos, wp, preferred_element_type=jnp.float32)
    o_ref[...] = (acc + const_ref[...]).reshape(rows, w, DM)


def kernel(grid, color_table, pos_emb, size_W, size_b, combine_W, combine_b):
    h, w = grid.shape
    nblk = 6
    rows = h // nblk
    ct_pad = jnp.pad(color_table, ((0, DQ - color_table.shape[0]), (0, 0)))
    full = lambda *s: pl.BlockSpec(s, lambda i: (0,) * len(s))
    return pl.pallas_call(
        functools.partial(_tc_body, h=h, w=w, rows=rows),
        grid=(nblk,),
        in_specs=[
            pl.BlockSpec((1, rows, w), lambda i: (i, 0, 0)),
            full(DQ, DQ),
            pl.BlockSpec((rows, w, DQ), lambda i: (i, 0, 0)),
            full(2, DQ),
            full(1, DQ),
            full(3 * DQ, DM),
            full(1, DM),
        ],
        out_specs=pl.BlockSpec((rows, w, DM), lambda i: (i, 0, 0)),
        out_shape=jax.ShapeDtypeStruct((h, w, DM), jnp.float32),
        scratch_shapes=[
            pltpu.VMEM((DQ, DM), jnp.float32),
            pltpu.VMEM((1, DM), jnp.float32),
        ],
    )(
        grid.reshape(nblk, rows, w).astype(jnp.int32),
        ct_pad,
        pos_emb[:h, :w],
        size_W,
        size_b.reshape(1, DQ),
        combine_W,
        combine_b.reshape(1, DM),
    )


# speedup vs baseline: 2.3620x; 2.3620x over previous
import functools

import jax
import jax.numpy as jnp
from jax.experimental import pallas as pl

DQ = 128
DM = 512


def _tc_full(idx_ref, ct_ref, p_ref, sw_ref, sb_ref, w_ref, b_ref,
             o_ref, *, h, w):
    n = h * w
    nc = ct_ref.shape[0]
    wc = w_ref[0:DQ, :]
    wp = w_ref[DQ:2 * DQ, :]
    ws = w_ref[2 * DQ:3 * DQ, :]
    size_e = float(h) * sw_ref[0:1, :] + float(w) * sw_ref[1:2, :] + sb_ref[0:1, :]
    const = jnp.dot(size_e, ws, preferred_element_type=jnp.float32) + b_ref[0:1, :]
    zt = jnp.dot(ct_ref[...], wc, preferred_element_type=jnp.float32)
    zt = jnp.concatenate([zt, jnp.zeros((DQ - nc, DM), jnp.float32)], axis=0)
    lanes = jax.lax.broadcasted_iota(jnp.int32, (h, w, DQ), 2)
    oh = (lanes == idx_ref[...][:, :, None]).astype(jnp.float32).reshape(n, DQ)
    acc = jnp.dot(oh, zt, preferred_element_type=jnp.float32)
    pos = p_ref[...].reshape(n, DQ)
    acc = acc + jnp.dot(pos, wp, preferred_element_type=jnp.float32)
    o_ref[...] = (acc + const).reshape(h, w, DM)


def kernel(grid, color_table, pos_emb, size_W, size_b, combine_W, combine_b):
    h, w = grid.shape
    return pl.pallas_call(
        functools.partial(_tc_full, h=h, w=w),
        out_shape=jax.ShapeDtypeStruct((h, w, DM), jnp.float32),
    )(
        grid.astype(jnp.int32),
        color_table,
        pos_emb[:h, :w],
        size_W,
        size_b.reshape(1, DQ),
        combine_W,
        combine_b.reshape(1, DM),
    )
